# baseline (speedup 1.0000x reference)
import jax
import jax.numpy as jnp
from jax.experimental import pallas as pl

N = 50000
F = 128
T = 32
S = 32
B = 128

BLOCK_N = 2000
NB = N // BLOCK_N


def _ect_kernel(x_ref, v_ref, batch_ref, lin_ref, out_ref, cnt_ref):
    i = pl.program_id(0)

    @pl.when(i == 0)
    def _():
        out_ref[...] = jnp.zeros_like(out_ref)
        cnt_ref[...] = jnp.zeros_like(cnt_ref)

    xb = x_ref[...]
    vv = v_ref[...]
    nht = jax.lax.dot_general(
        vv, xb, (((0,), (1,)), ((), ())), preferred_element_type=jnp.float32
    )
    b2 = 100.0 * nht
    bflat = jnp.broadcast_to(b2[None, :, :], (S, T, BLOCK_N)).reshape(S * T, BLOCK_N)
    z = lin_ref[...] - bflat
    th = jnp.tanh(z).astype(jnp.bfloat16)

    bcol = batch_ref[0]
    iota = jax.lax.broadcasted_iota(jnp.int32, (BLOCK_N, B), 1).astype(jnp.float32)
    onehot = (iota == bcol).astype(jnp.float32)

    out_ref[...] += jnp.dot(th, onehot.astype(jnp.bfloat16),
                            preferred_element_type=jnp.float32)
    cnt_ref[...] += jnp.sum(onehot, axis=0, keepdims=True)


@jax.jit
def kernel(x, batch, v, lin):
    lin_col = 100.0 * jnp.broadcast_to(lin.reshape(S, 1, 1), (S, T, 1)).reshape(S * T, 1)
    batch_col = batch.astype(jnp.float32).reshape(NB, BLOCK_N, 1)

    out, cnt = pl.pallas_call(
        _ect_kernel,
        grid=(NB,),
        in_specs=[
            pl.BlockSpec((BLOCK_N, F), lambda i: (i, 0)),
            pl.BlockSpec((F, T), lambda i: (0, 0)),
            pl.BlockSpec((1, BLOCK_N, 1), lambda i: (i, 0, 0)),
            pl.BlockSpec((S * T, 1), lambda i: (0, 0)),
        ],
        out_specs=[
            pl.BlockSpec((S * T, B), lambda i: (0, 0)),
            pl.BlockSpec((1, B), lambda i: (0, 0)),
        ],
        out_shape=[
            jax.ShapeDtypeStruct((S * T, B), jnp.float32),
            jax.ShapeDtypeStruct((1, B), jnp.float32),
        ],
    )(x, v, batch_col, lin_col)

    return (0.5 * (out + cnt)).T.reshape(B, S, T)


# speedup vs baseline: 1.1245x; 1.1245x over previous
import jax
import jax.numpy as jnp
from jax.experimental import pallas as pl
from jax.experimental.pallas import tpu as pltpu

N = 50000
F = 128
T = 32
S = 32
B = 128

BLOCK_N = 2000
NB = N // BLOCK_N


def _ect_kernel(x_ref, v_ref, batch_ref, lin_ref, out_ref, acc_ref, cnt_ref):
    i = pl.program_id(0)

    @pl.when(i == 0)
    def _():
        acc_ref[...] = jnp.zeros_like(acc_ref)
        cnt_ref[...] = jnp.zeros_like(cnt_ref)

    xb = x_ref[...]
    vv = v_ref[...]
    nht = jax.lax.dot_general(
        vv, xb, (((0,), (1,)), ((), ())), preferred_element_type=jnp.float32
    )
    b2 = 100.0 * nht
    bflat = jnp.broadcast_to(b2[None, :, :], (S, T, BLOCK_N)).reshape(S * T, BLOCK_N)
    z = lin_ref[...] - bflat
    th = jnp.tanh(z)

    bcol = batch_ref[0].astype(jnp.float32)
    iota = jax.lax.broadcasted_iota(jnp.int32, (BLOCK_N, B), 1).astype(jnp.float32)
    onehot = (iota == bcol).astype(jnp.float32)

    acc_ref[...] += jnp.dot(th, onehot, preferred_element_type=jnp.float32)
    cnt_ref[0:1, :] += jnp.sum(onehot, axis=0, keepdims=True)

    @pl.when(i == NB - 1)
    def _():
        acc_t = acc_ref[...].T
        cnt_t = cnt_ref[0:1, :].T
        out_ref[...] = 0.5 * (acc_t + cnt_t)


@jax.jit
def kernel(x, batch, v, lin):
    lin_col = 100.0 * jnp.broadcast_to(lin.reshape(S, 1, 1), (S, T, 1)).reshape(S * T, 1)
    batch_col = batch.reshape(NB, BLOCK_N, 1)

    out = pl.pallas_call(
        _ect_kernel,
        grid=(NB,),
        in_specs=[
            pl.BlockSpec((BLOCK_N, F), lambda i: (i, 0)),
            pl.BlockSpec((F, T), lambda i: (0, 0)),
            pl.BlockSpec((1, BLOCK_N, 1), lambda i: (i, 0, 0)),
            pl.BlockSpec((S * T, 1), lambda i: (0, 0)),
        ],
        out_specs=pl.BlockSpec((B, S * T), lambda i: (0, 0)),
        out_shape=jax.ShapeDtypeStruct((B, S * T), jnp.float32),
        scratch_shapes=[
            pltpu.VMEM((S * T, B), jnp.float32),
            pltpu.VMEM((8, B), jnp.float32),
        ],
    )(x, v, batch_col, lin_col)

    return out.reshape(B, S, T)


# speedup vs baseline: 1.1655x; 1.0365x over previous
import jax
import jax.numpy as jnp
from jax.experimental import pallas as pl
from jax.experimental.pallas import tpu as pltpu

N = 50000
F = 128
T = 32
S = 32
B = 128

BLOCK_N = 5000
NB = N // BLOCK_N


def _ect_kernel(x_ref, v_ref, batch_ref, lin_ref, out_ref, acc_ref, cnt_ref):
    i = pl.program_id(0)

    @pl.when(i == 0)
    def _():
        acc_ref[...] = jnp.zeros_like(acc_ref)
        cnt_ref[...] = jnp.zeros_like(cnt_ref)

    xb = x_ref[...]
    vv = v_ref[...]
    nht = jax.lax.dot_general(
        vv, xb, (((0,), (1,)), ((), ())), preferred_element_type=jnp.float32
    )
    b2 = 100.0 * nht
    bflat = jnp.broadcast_to(b2[None, :, :], (S, T, BLOCK_N)).reshape(S * T, BLOCK_N)
    z = lin_ref[...] - bflat
    th = jnp.tanh(z)

    bcol = batch_ref[0].astype(jnp.float32)
    iota = jax.lax.broadcasted_iota(jnp.int32, (BLOCK_N, B), 1).astype(jnp.float32)
    onehot = (iota == bcol).astype(jnp.float32)

    acc_ref[...] += jnp.dot(th, onehot, preferred_element_type=jnp.float32)
    cnt_ref[0:1, :] += jnp.sum(onehot, axis=0, keepdims=True)

    @pl.when(i == NB - 1)
    def _():
        acc_t = acc_ref[...].T
        cnt_t = cnt_ref[0:1, :].T
        out_ref[...] = 0.5 * (acc_t + cnt_t)


@jax.jit
def kernel(x, batch, v, lin):
    lin_col = 100.0 * jnp.broadcast_to(lin.reshape(S, 1, 1), (S, T, 1)).reshape(S * T, 1)
    batch_col = batch.reshape(NB, BLOCK_N, 1)

    out = pl.pallas_call(
        _ect_kernel,
        grid=(NB,),
        in_specs=[
            pl.BlockSpec((BLOCK_N, F), lambda i: (i, 0)),
            pl.BlockSpec((F, T), lambda i: (0, 0)),
            pl.BlockSpec((1, BLOCK_N, 1), lambda i: (i, 0, 0)),
            pl.BlockSpec((S * T, 1), lambda i: (0, 0)),
        ],
        out_specs=pl.BlockSpec((B, S * T), lambda i: (0, 0)),
        out_shape=jax.ShapeDtypeStruct((B, S * T), jnp.float32),
        scratch_shapes=[
            pltpu.VMEM((S * T, B), jnp.float32),
            pltpu.VMEM((8, B), jnp.float32),
        ],
    )(x, v, batch_col, lin_col)

    return out.reshape(B, S, T)
